# baseline (speedup 1.0000x reference)
import jax
import jax.numpy as jnp
from jax.experimental import pallas as pl
from jax.experimental.pallas import tpu as pltpu

_NUM_CHUNKS = 4


def _copy_body(emb_hbm, o_hbm):
    rows = o_hbm.shape[0]
    chunk = rows // _NUM_CHUNKS

    def scoped(sems):
        copies = [
            pltpu.make_async_copy(
                emb_hbm.at[pl.ds(i * chunk, chunk), :],
                o_hbm.at[pl.ds(i * chunk, chunk), :],
                sems.at[i],
            )
            for i in range(_NUM_CHUNKS)
        ]
        for c in copies:
            c.start()
        for c in copies:
            c.wait()

    pl.run_scoped(scoped, pltpu.SemaphoreType.DMA((_NUM_CHUNKS,)))


def kernel(x, emb):
    seq_len = x.shape[1]
    dim = emb.shape[1]
    return pl.pallas_call(
        _copy_body,
        in_specs=[pl.BlockSpec(memory_space=pl.ANY)],
        out_specs=pl.BlockSpec(memory_space=pl.ANY),
        out_shape=jax.ShapeDtypeStruct((seq_len, dim), emb.dtype),
    )(emb)


# speedup vs baseline: 5.7144x; 5.7144x over previous
import functools

import jax
import jax.numpy as jnp
from jax import lax
from jax.experimental import pallas as pl
from jax.experimental.pallas import tpu as pltpu
from jax.experimental.pallas import tpu_sc as plsc


def kernel(x, emb):
    seq_len = x.shape[1]
    dim = emb.shape[1]
    info = plsc.get_sparse_core_info()
    num_workers = info.num_cores * info.num_subcores
    rows_per_w = seq_len // num_workers
    mesh = plsc.VectorSubcoreMesh(core_axis_name="c", subcore_axis_name="s")

    @functools.partial(
        pl.kernel,
        mesh=mesh,
        out_type=jax.ShapeDtypeStruct((seq_len, dim), emb.dtype),
        scratch_types=[pltpu.VMEM((rows_per_w, dim), jnp.float32)],
    )
    def copy_k(emb_hbm, out_hbm, rows_v):
        wid = lax.axis_index("s") * info.num_cores + lax.axis_index("c")
        base = wid * rows_per_w
        pltpu.sync_copy(emb_hbm.at[pl.ds(base, rows_per_w), :], rows_v)
        pltpu.sync_copy(rows_v, out_hbm.at[pl.ds(base, rows_per_w), :])

    return copy_k(emb)


# speedup vs baseline: 35.4710x; 6.2072x over previous
import jax
import jax.numpy as jnp
from jax.experimental import pallas as pl
from jax.experimental.pallas import tpu as pltpu

_NUM_CHUNKS = 8


def _make_body(seq_len, dim):
    chunk = seq_len // _NUM_CHUNKS

    def body(emb_hbm, o_hbm, scratch, in_sems, out_sems):
        reads = [
            pltpu.make_async_copy(
                emb_hbm.at[pl.ds(i * chunk, chunk), :],
                scratch.at[pl.ds(i * chunk, chunk), :],
                in_sems.at[i],
            )
            for i in range(_NUM_CHUNKS)
        ]
        writes = [
            pltpu.make_async_copy(
                scratch.at[pl.ds(i * chunk, chunk), :],
                o_hbm.at[pl.ds(i * chunk, chunk), :],
                out_sems.at[i],
            )
            for i in range(_NUM_CHUNKS)
        ]
        for r in reads:
            r.start()
        for r, w in zip(reads, writes):
            r.wait()
            w.start()
        for w in writes:
            w.wait()

    return body


def kernel(x, emb):
    seq_len = x.shape[1]
    dim = emb.shape[1]
    return pl.pallas_call(
        _make_body(seq_len, dim),
        in_specs=[pl.BlockSpec(memory_space=pl.ANY)],
        out_specs=pl.BlockSpec(memory_space=pl.ANY),
        out_shape=jax.ShapeDtypeStruct((seq_len, dim), emb.dtype),
        scratch_shapes=[
            pltpu.VMEM((seq_len, dim), emb.dtype),
            pltpu.SemaphoreType.DMA((_NUM_CHUNKS,)),
            pltpu.SemaphoreType.DMA((_NUM_CHUNKS,)),
        ],
    )(emb)
